# baseline (speedup 1.0000x reference)
import functools

import jax
import jax.numpy as jnp
from jax import lax
from jax.experimental import pallas as pl
from jax.experimental.pallas import tpu as pltpu
from jax.experimental.pallas import tpu_sc as plsc

_N = 16384
_C = 1000
_CP = 1024
_GAMMA0 = 2.0
_ALPHA_R = 0.1
_TAU = 0.2
_LAMBDA_MAX = 1.0
_TOTAL_EPOCHS = 200
_EMA = 0.9
_EPS = 1e-6
_HEAD = 100

_ROWS_BLK = 1024
_GRID = _N // _ROWS_BLK

_NSUB = 16
_RPT = _N // _NSUB
_CPT = _CP // _NSUB


def _row_stats(x, t):
    m = jnp.max(x, axis=1, keepdims=True)
    e = jnp.exp(x - m)
    s = jnp.sum(e, axis=1, keepdims=True)
    xs = jnp.sum(e * x, axis=1, keepdims=True)
    col = lax.broadcasted_iota(jnp.int32, x.shape, 1)
    et = jnp.sum(jnp.where(col == t, e, 0.0), axis=1, keepdims=True)
    pt = et / s
    return -jnp.log(pt + _EPS), m + jnp.log(s) - xs / s, jnp.log(1.0 - pt)


def _tc_stats_body(xa_ref, xb_ref, t_ref, lce_ref, ent_ref, l1p_ref):
    la, ea, pa = _row_stats(xa_ref[...], t_ref[pl.ds(0, _ROWS_BLK), :])
    lb, eb, pb = _row_stats(xb_ref[...], t_ref[pl.ds(_ROWS_BLK, _ROWS_BLK), :])
    lce_ref[...] = jnp.concatenate([la, lb], axis=0)
    ent_ref[...] = jnp.concatenate([ea, eb], axis=0)
    l1p_ref[...] = jnp.concatenate([pa, pb], axis=0)


def _tc_stats(logits, targets2d):
    blk2 = 2 * _ROWS_BLK
    return pl.pallas_call(
        _tc_stats_body,
        grid=(_N // blk2,),
        in_specs=[
            pl.BlockSpec((_ROWS_BLK, _C), lambda i: (2 * i, 0)),
            pl.BlockSpec((_ROWS_BLK, _C), lambda i: (2 * i + 1, 0)),
            pl.BlockSpec((blk2, 1), lambda i: (i, 0)),
        ],
        out_specs=[
            pl.BlockSpec((blk2, 1), lambda i: (i, 0)),
            pl.BlockSpec((blk2, 1), lambda i: (i, 0)),
            pl.BlockSpec((blk2, 1), lambda i: (i, 0)),
        ],
        out_shape=[
            jax.ShapeDtypeStruct((_N, 1), jnp.float32),
            jax.ShapeDtypeStruct((_N, 1), jnp.float32),
            jax.ShapeDtypeStruct((_N, 1), jnp.float32),
        ],
    )(logits, logits, targets2d)


def _sc_body(t_hbm, lce_hbm, ent_hbm, l1p_hbm, lavg_hbm, eavg_hbm, wrl_hbm,
             lam_hbm, out_hbm,
             tgt_v, lce_v, ent_v, l1p_v,
             acc3_v,
             oldl_v, olde_v, tmp48_v, resl_v, resi_v,
             lavg_v, invh_v, wrl_v, lam_v, psum_v, sums_v, rsums_v, out_v,
             dma_sem,
             acc_sh, statl_sh, stati_sh, sum_sh, rsum_sh):
    cid = lax.axis_index("c")
    sid = lax.axis_index("s")
    base = sid * _RPT

    stage = [
        pltpu.async_copy(t_hbm.at[pl.ds(base, _RPT)], tgt_v, dma_sem),
        pltpu.async_copy(lce_hbm.at[pl.ds(base, _RPT)], lce_v, dma_sem),
        pltpu.async_copy(ent_hbm.at[pl.ds(base, _RPT)], ent_v, dma_sem),
        pltpu.async_copy(l1p_hbm.at[pl.ds(base, _RPT)], l1p_v, dma_sem),
        pltpu.async_copy(wrl_hbm, wrl_v.at[pl.ds(0, _C)], dma_sem),
        pltpu.async_copy(lam_hbm, lam_v, dma_sem),
    ]
    for c in stage:
        c.wait()

    zero16 = jnp.zeros((16,), jnp.float32)
    ones16 = jnp.ones((16,), jnp.float32)

    def _zero_body(i, carry):
        acc3_v[pl.ds(i * 16, 16)] = zero16
        return carry

    lax.fori_loop(0, 3 * _CP // 16, _zero_body, 0)

    def _scat_body(i, carry):
        sl = pl.ds(i * 16, 16)
        idx = tgt_v[sl]
        plsc.addupdate_scatter(acc3_v, [idx], ones16)
        plsc.addupdate_scatter(acc3_v, [idx + _CP], lce_v[sl])
        plsc.addupdate_scatter(acc3_v, [idx + 2 * _CP], ent_v[sl])
        return carry

    lax.fori_loop(0, _RPT // 16, _scat_body, 0)

    pltpu.sync_copy(acc3_v, acc_sh.at[pl.ds(sid * (3 * _CP), 3 * _CP)])
    plsc.subcore_barrier()

    cbase = sid * _CPT
    copies = []
    for src in range(_NSUB):
        for r in range(3):
            copies.append(pltpu.async_copy(
                acc_sh.at[pl.ds(src * (3 * _CP) + r * _CP + cbase, _CPT)],
                tmp48_v.at[pl.ds((src * 3 + r) * _CPT, _CPT)],
                dma_sem))
    for c in copies:
        c.wait()

    nk = _CPT // 16
    cnt_a = [zero16] * nk
    lsum_a = [zero16] * nk
    esum_a = [zero16] * nk
    for src in range(_NSUB):
        for k in range(nk):
            b = src * 3 * _CPT + k * 16
            cnt_a[k] = cnt_a[k] + tmp48_v[pl.ds(b, 16)]
            lsum_a[k] = lsum_a[k] + tmp48_v[pl.ds(b + _CPT, 16)]
            esum_a[k] = esum_a[k] + tmp48_v[pl.ds(b + 2 * _CPT, 16)]

    @pl.when(sid < _NSUB - 1)
    def _():
        pltpu.sync_copy(lavg_hbm.at[pl.ds(cbase, _CPT)], oldl_v)
        pltpu.sync_copy(eavg_hbm.at[pl.ds(cbase, _CPT)], olde_v)

    @pl.when(sid == _NSUB - 1)
    def _():
        tail = _C - (_NSUB - 1) * _CPT
        pltpu.sync_copy(lavg_hbm.at[pl.ds(cbase, tail)], oldl_v.at[pl.ds(0, tail)])
        pltpu.sync_copy(eavg_hbm.at[pl.ds(cbase, tail)], olde_v.at[pl.ds(0, tail)])

    part = zero16
    for k in range(nk):
        sl = pl.ds(k * 16, 16)
        cnt = cnt_a[k]
        lsum = lsum_a[k]
        esum = esum_a[k]
        old_l = oldl_v[sl]
        old_e = olde_v[sl]
        present = cnt > 0.0
        denom = jnp.maximum(cnt, 1.0)
        new_l = jnp.where(present, _EMA * old_l + (1.0 - _EMA) * (lsum / denom), old_l)
        new_e = jnp.where(present, _EMA * old_e + (1.0 - _EMA) * (esum / denom), old_e)
        invh = 1.0 / (new_e + _EPS)
        cids = (cbase + k * 16) + lax.iota(jnp.int32, 16)
        invh = jnp.where(cids < _C, invh, 0.0)
        resl_v[sl] = new_l
        resi_v[sl] = invh
        part = part + invh

    psum_v[...] = jnp.broadcast_to(jnp.sum(part), (16,))
    pltpu.sync_copy(resl_v, statl_sh.at[pl.ds(cbase, _CPT)])
    pltpu.sync_copy(resi_v, stati_sh.at[pl.ds(cbase, _CPT)])
    pltpu.sync_copy(psum_v, sum_sh.at[pl.ds(sid * 16, 16)])
    plsc.subcore_barrier()

    stage3 = [
        pltpu.async_copy(statl_sh, lavg_v, dma_sem),
        pltpu.async_copy(stati_sh, invh_v, dma_sem),
        pltpu.async_copy(sum_sh, sums_v, dma_sem),
    ]
    for c in stage3:
        c.wait()
    tot = zero16
    for src in range(_NSUB):
        tot = tot + sums_v[pl.ds(src * 16, 16)]
    inv_tot = 1.0 / tot
    lamv = lam_v[...]

    def _row_body(i, acc):
        sl = pl.ds(i * 16, 16)
        t16 = tgt_v[sl]
        lce16 = lce_v[sl]
        l1p16 = l1p_v[sl]
        avg_c = plsc.load_gather(lavg_v, [t16])
        invh_t = plsc.load_gather(invh_v, [t16])
        wrl_t = plsc.load_gather(wrl_v, [t16])
        gamma = _GAMMA0 * lce16 / (avg_c + _EPS)
        focal = jnp.exp(gamma * l1p16)
        went_t = invh_t * inv_tot
        red = (lce16 < _TAU) & (lce16 >= lamv) & (t16 < _HEAD)
        wsp = jnp.where(lce16 < lamv, 0.0, jnp.where(red, _ALPHA_R, 1.0))
        wcomb = (wsp + _EPS) * (went_t + _EPS) * (wrl_t + _EPS)
        return acc + wcomb * focal * lce16

    acc = lax.fori_loop(0, _RPT // 16, _row_body, zero16)
    psum_v[...] = acc
    pltpu.sync_copy(psum_v, rsum_sh.at[pl.ds(sid * 16, 16)])
    plsc.subcore_barrier()

    @pl.when((cid == 0) & (sid == 0))
    def _():
        pltpu.sync_copy(rsum_sh, rsums_v)
        tot2 = zero16
        for src in range(_NSUB):
            tot2 = tot2 + rsums_v[pl.ds(src * 16, 16)]
        out_v[...] = jnp.broadcast_to(jnp.sum(tot2) * (1.0 / _N), (16,))
        pltpu.sync_copy(out_v, out_hbm)


def _make_sc_combine():
  return functools.partial(
    pl.kernel,
    out_type=jax.ShapeDtypeStruct((16,), jnp.float32),
    mesh=plsc.VectorSubcoreMesh(core_axis_name="c", subcore_axis_name="s"),
    compiler_params=pltpu.CompilerParams(needs_layout_passes=False),
    scratch_types=[
        pltpu.VMEM((_RPT,), jnp.int32),
        pltpu.VMEM((_RPT,), jnp.float32),
        pltpu.VMEM((_RPT,), jnp.float32),
        pltpu.VMEM((_RPT,), jnp.float32),
        pltpu.VMEM((3 * _CP,), jnp.float32),
        pltpu.VMEM((_CPT,), jnp.float32),
        pltpu.VMEM((_CPT,), jnp.float32),
        pltpu.VMEM((_NSUB * 3 * _CPT,), jnp.float32),
        pltpu.VMEM((_CPT,), jnp.float32),
        pltpu.VMEM((_CPT,), jnp.float32),
        pltpu.VMEM((_CP,), jnp.float32),
        pltpu.VMEM((_CP,), jnp.float32),
        pltpu.VMEM((_CP,), jnp.float32),
        pltpu.VMEM((16,), jnp.float32),
        pltpu.VMEM((16,), jnp.float32),
        pltpu.VMEM((_NSUB * 16,), jnp.float32),
        pltpu.VMEM((_NSUB * 16,), jnp.float32),
        pltpu.VMEM((16,), jnp.float32),
        pltpu.SemaphoreType.DMA,
        pltpu.VMEM_SHARED((_NSUB * 3 * _CP,), jnp.float32),
        pltpu.VMEM_SHARED((_CP,), jnp.float32),
        pltpu.VMEM_SHARED((_CP,), jnp.float32),
        pltpu.VMEM_SHARED((_NSUB * 16,), jnp.float32),
        pltpu.VMEM_SHARED((_NSUB * 16,), jnp.float32),
    ],
  )(_sc_body)


def kernel(logits, targets, epoch, class_loss_avg, class_entropy_avg,
           class_rl_weight):
    lce, ent, l1p = _tc_stats(logits, targets.reshape(_N, 1))
    lce = lce.reshape(_N)
    ent = ent.reshape(_N)
    l1p = l1p.reshape(_N)

    ep = jnp.asarray(epoch, jnp.float32)
    lam = _LAMBDA_MAX * jnp.minimum(1.0, ep / _TOTAL_EPOCHS)
    lam16 = jnp.broadcast_to(lam.astype(jnp.float32), (16,))

    out16 = _make_sc_combine()(targets, lce, ent, l1p, class_loss_avg,
                               class_entropy_avg, class_rl_weight, lam16)
    return out16[0]


# speedup vs baseline: 1.0284x; 1.0284x over previous
import functools

import jax
import jax.numpy as jnp
from jax import lax
from jax.experimental import pallas as pl
from jax.experimental.pallas import tpu as pltpu
from jax.experimental.pallas import tpu_sc as plsc

_N = 16384
_C = 1000
_CP = 1024
_GAMMA0 = 2.0
_ALPHA_R = 0.1
_TAU = 0.2
_LAMBDA_MAX = 1.0
_TOTAL_EPOCHS = 200
_EMA = 0.9
_EPS = 1e-6
_HEAD = 100

_ROWS_BLK = 1024
_GRID = _N // _ROWS_BLK

_NSUB = 16
_RPT = _N // _NSUB
_CPT = _CP // _NSUB


def _row_stats(x, t):
    m = jnp.max(x, axis=1, keepdims=True)
    e = jnp.exp(x - m)
    s = jnp.sum(e, axis=1, keepdims=True)
    xs = jnp.sum(e * x, axis=1, keepdims=True)
    col = lax.broadcasted_iota(jnp.int32, x.shape, 1)
    h = jnp.where(col == t, 1.0, 0.0)
    et = jnp.sum(h * e, axis=1, keepdims=True)
    pt = et / s
    lce = -jnp.log(pt + _EPS)
    ent = m + jnp.log(s) - xs / s
    return lce, ent, jnp.log(1.0 - pt), h


def _tc_stats_body(xa_ref, xb_ref, t_ref, lce_ref, l1p_ref,
                   cnt_ref, lsm_ref, esm_ref):
    i = pl.program_id(0)
    la, ea, pa, ha = _row_stats(xa_ref[...], t_ref[pl.ds(0, _ROWS_BLK), :])
    lb, eb, pb, hb = _row_stats(xb_ref[...], t_ref[pl.ds(_ROWS_BLK, _ROWS_BLK), :])
    lce_ref[...] = jnp.concatenate([la, lb], axis=0).reshape(2 * _ROWS_BLK)
    l1p_ref[...] = jnp.concatenate([pa, pb], axis=0).reshape(2 * _ROWS_BLK)
    ones = jnp.ones_like(la)
    va = jnp.concatenate([ones, la, ea], axis=1)
    vb = jnp.concatenate([ones, lb, eb], axis=1)
    dims = (((0,), (0,)), ((), ()))
    seg = (lax.dot_general(va, ha, dims, preferred_element_type=jnp.float32)
           + lax.dot_general(vb, hb, dims, preferred_element_type=jnp.float32))

    @pl.when(i == 0)
    def _():
        cnt_ref[...] = jnp.zeros((_C,), jnp.float32)
        lsm_ref[...] = jnp.zeros((_C,), jnp.float32)
        esm_ref[...] = jnp.zeros((_C,), jnp.float32)

    cnt_ref[...] = cnt_ref[...] + seg[0, :]
    lsm_ref[...] = lsm_ref[...] + seg[1, :]
    esm_ref[...] = esm_ref[...] + seg[2, :]


def _tc_stats(logits, targets2d):
    blk2 = 2 * _ROWS_BLK
    return pl.pallas_call(
        _tc_stats_body,
        grid=(_N // blk2,),
        in_specs=[
            pl.BlockSpec((_ROWS_BLK, _C), lambda i: (2 * i, 0)),
            pl.BlockSpec((_ROWS_BLK, _C), lambda i: (2 * i + 1, 0)),
            pl.BlockSpec((blk2, 1), lambda i: (i, 0)),
        ],
        out_specs=[
            pl.BlockSpec((blk2,), lambda i: (i,)),
            pl.BlockSpec((blk2,), lambda i: (i,)),
            pl.BlockSpec((_C,), lambda i: (0,)),
            pl.BlockSpec((_C,), lambda i: (0,)),
            pl.BlockSpec((_C,), lambda i: (0,)),
        ],
        out_shape=[
            jax.ShapeDtypeStruct((_N,), jnp.float32),
            jax.ShapeDtypeStruct((_N,), jnp.float32),
            jax.ShapeDtypeStruct((_C,), jnp.float32),
            jax.ShapeDtypeStruct((_C,), jnp.float32),
            jax.ShapeDtypeStruct((_C,), jnp.float32),
        ],
    )(logits, logits, targets2d)


def _sc_body(t_hbm, lce_hbm, l1p_hbm, cnts_hbm, lsms_hbm, esms_hbm,
             lavg_hbm, eavg_hbm, wrl_hbm,
             lam_hbm, out_hbm,
             tgt_v, lce_v, l1p_v,
             cnt_v, lsm_v, esm_v,
             oldl_v, olde_v, resl_v, resi_v,
             lavg_v, invh_v, wrl_v, lam_v, psum_v, sums_v, rsums_v, out_v,
             dma_sem,
             statl_sh, stati_sh, sum_sh, rsum_sh):
    cid = lax.axis_index("c")
    sid = lax.axis_index("s")
    base = sid * _RPT

    stage = [
        pltpu.async_copy(t_hbm.at[pl.ds(base, _RPT)], tgt_v, dma_sem),
        pltpu.async_copy(lce_hbm.at[pl.ds(base, _RPT)], lce_v, dma_sem),
        pltpu.async_copy(l1p_hbm.at[pl.ds(base, _RPT)], l1p_v, dma_sem),
        pltpu.async_copy(wrl_hbm, wrl_v.at[pl.ds(0, _C)], dma_sem),
        pltpu.async_copy(lam_hbm, lam_v, dma_sem),
    ]

    zero16 = jnp.zeros((16,), jnp.float32)
    nk = _CPT // 16

    cbase = sid * _CPT
    for k in range(nk):
        sl = pl.ds(k * 16, 16)
        cnt_v[sl] = zero16
        lsm_v[sl] = zero16
        esm_v[sl] = zero16
        oldl_v[sl] = zero16
        olde_v[sl] = zero16

    @pl.when(sid < _NSUB - 1)
    def _():
        pltpu.sync_copy(cnts_hbm.at[pl.ds(cbase, _CPT)], cnt_v)
        pltpu.sync_copy(lsms_hbm.at[pl.ds(cbase, _CPT)], lsm_v)
        pltpu.sync_copy(esms_hbm.at[pl.ds(cbase, _CPT)], esm_v)
        pltpu.sync_copy(lavg_hbm.at[pl.ds(cbase, _CPT)], oldl_v)
        pltpu.sync_copy(eavg_hbm.at[pl.ds(cbase, _CPT)], olde_v)

    @pl.when(sid == _NSUB - 1)
    def _():
        tail = _C - (_NSUB - 1) * _CPT
        pltpu.sync_copy(cnts_hbm.at[pl.ds(cbase, tail)], cnt_v.at[pl.ds(0, tail)])
        pltpu.sync_copy(lsms_hbm.at[pl.ds(cbase, tail)], lsm_v.at[pl.ds(0, tail)])
        pltpu.sync_copy(esms_hbm.at[pl.ds(cbase, tail)], esm_v.at[pl.ds(0, tail)])
        pltpu.sync_copy(lavg_hbm.at[pl.ds(cbase, tail)], oldl_v.at[pl.ds(0, tail)])
        pltpu.sync_copy(eavg_hbm.at[pl.ds(cbase, tail)], olde_v.at[pl.ds(0, tail)])

    part = zero16
    for k in range(nk):
        sl = pl.ds(k * 16, 16)
        cnt = cnt_v[sl]
        lsum = lsm_v[sl]
        esum = esm_v[sl]
        old_l = oldl_v[sl]
        old_e = olde_v[sl]
        present = cnt > 0.0
        denom = jnp.maximum(cnt, 1.0)
        new_l = jnp.where(present, _EMA * old_l + (1.0 - _EMA) * (lsum / denom), old_l)
        new_e = jnp.where(present, _EMA * old_e + (1.0 - _EMA) * (esum / denom), old_e)
        invh = 1.0 / (new_e + _EPS)
        cids = (cbase + k * 16) + lax.iota(jnp.int32, 16)
        invh = jnp.where(cids < _C, invh, 0.0)
        resl_v[sl] = new_l
        resi_v[sl] = invh
        part = part + invh

    psum_v[...] = jnp.broadcast_to(jnp.sum(part), (16,))
    pltpu.sync_copy(resl_v, statl_sh.at[pl.ds(cbase, _CPT)])
    pltpu.sync_copy(resi_v, stati_sh.at[pl.ds(cbase, _CPT)])
    pltpu.sync_copy(psum_v, sum_sh.at[pl.ds(sid * 16, 16)])
    plsc.subcore_barrier()

    stage3 = [
        pltpu.async_copy(statl_sh, lavg_v, dma_sem),
        pltpu.async_copy(stati_sh, invh_v, dma_sem),
        pltpu.async_copy(sum_sh, sums_v, dma_sem),
    ]
    for c in stage:
        c.wait()
    for c in stage3:
        c.wait()
    tot = zero16
    for src in range(_NSUB):
        tot = tot + sums_v[pl.ds(src * 16, 16)]
    inv_tot = 1.0 / tot
    lamv = lam_v[...]

    def _row_body(i, acc):
        sl = pl.ds(i * 16, 16)
        t16 = tgt_v[sl]
        lce16 = lce_v[sl]
        l1p16 = l1p_v[sl]
        avg_c = plsc.load_gather(lavg_v, [t16])
        invh_t = plsc.load_gather(invh_v, [t16])
        wrl_t = plsc.load_gather(wrl_v, [t16])
        gamma = _GAMMA0 * lce16 / (avg_c + _EPS)
        focal = jnp.exp(gamma * l1p16)
        went_t = invh_t * inv_tot
        red = (lce16 < _TAU) & (lce16 >= lamv) & (t16 < _HEAD)
        wsp = jnp.where(lce16 < lamv, 0.0, jnp.where(red, _ALPHA_R, 1.0))
        wcomb = (wsp + _EPS) * (went_t + _EPS) * (wrl_t + _EPS)
        return acc + wcomb * focal * lce16

    acc = lax.fori_loop(0, _RPT // 16, _row_body, zero16)
    psum_v[...] = acc
    pltpu.sync_copy(psum_v, rsum_sh.at[pl.ds(sid * 16, 16)])
    plsc.subcore_barrier()

    @pl.when((cid == 0) & (sid == 0))
    def _():
        pltpu.sync_copy(rsum_sh, rsums_v)
        tot2 = zero16
        for src in range(_NSUB):
            tot2 = tot2 + rsums_v[pl.ds(src * 16, 16)]
        out_v[...] = jnp.broadcast_to(jnp.sum(tot2) * (1.0 / _N), (16,))
        pltpu.sync_copy(out_v, out_hbm)


def _make_sc_combine():
  return functools.partial(
    pl.kernel,
    out_type=jax.ShapeDtypeStruct((16,), jnp.float32),
    mesh=plsc.VectorSubcoreMesh(core_axis_name="c", subcore_axis_name="s"),
    compiler_params=pltpu.CompilerParams(needs_layout_passes=False),
    scratch_types=[
        pltpu.VMEM((_RPT,), jnp.int32),
        pltpu.VMEM((_RPT,), jnp.float32),
        pltpu.VMEM((_RPT,), jnp.float32),
        pltpu.VMEM((_CPT,), jnp.float32),
        pltpu.VMEM((_CPT,), jnp.float32),
        pltpu.VMEM((_CPT,), jnp.float32),
        pltpu.VMEM((_CPT,), jnp.float32),
        pltpu.VMEM((_CPT,), jnp.float32),
        pltpu.VMEM((_CPT,), jnp.float32),
        pltpu.VMEM((_CPT,), jnp.float32),
        pltpu.VMEM((_CP,), jnp.float32),
        pltpu.VMEM((_CP,), jnp.float32),
        pltpu.VMEM((_CP,), jnp.float32),
        pltpu.VMEM((16,), jnp.float32),
        pltpu.VMEM((16,), jnp.float32),
        pltpu.VMEM((_NSUB * 16,), jnp.float32),
        pltpu.VMEM((_NSUB * 16,), jnp.float32),
        pltpu.VMEM((16,), jnp.float32),
        pltpu.SemaphoreType.DMA,
        pltpu.VMEM_SHARED((_CP,), jnp.float32),
        pltpu.VMEM_SHARED((_CP,), jnp.float32),
        pltpu.VMEM_SHARED((_NSUB * 16,), jnp.float32),
        pltpu.VMEM_SHARED((_NSUB * 16,), jnp.float32),
    ],
  )(_sc_body)


def kernel(logits, targets, epoch, class_loss_avg, class_entropy_avg,
           class_rl_weight):
    lce, l1p, cnts, lsms, esms = _tc_stats(logits, targets.reshape(_N, 1))

    ep = jnp.asarray(epoch, jnp.float32)
    lam = _LAMBDA_MAX * jnp.minimum(1.0, ep / _TOTAL_EPOCHS)
    lam16 = jnp.broadcast_to(lam.astype(jnp.float32), (16,))

    out16 = _make_sc_combine()(targets, lce, l1p, cnts, lsms, esms,
                               class_loss_avg, class_entropy_avg,
                               class_rl_weight, lam16)
    return out16[0]
